# Initial kernel scaffold; baseline (speedup 1.0000x reference)
import jax
import jax.numpy as jnp
from jax.experimental import pallas as pl
from jax.experimental.pallas import tpu as pltpu

_B, _S, _D, _FF, _E = 1, 2048, 1024, 4096, 8
_FT = 1024
_NFT = _FF // _FT


def _routing(logits):
    probs = jax.nn.softmax(logits, axis=-1)
    idx = jax.lax.broadcasted_iota(jnp.int32, probs.shape, 1)
    m1 = jnp.max(probs, axis=-1, keepdims=True)
    i1 = jnp.min(jnp.where(probs == m1, idx, _E), axis=-1, keepdims=True)
    masked = jnp.where(idx == i1, -1.0, probs)
    m2 = jnp.max(masked, axis=-1, keepdims=True)
    i2 = jnp.min(jnp.where(masked == m2, idx, _E), axis=-1, keepdims=True)
    s = m1 + m2
    return jnp.where(idx == i1, m1 / s, 0.0) + jnp.where(idx == i2, m2 / s, 0.0)


def _moe_dense(x_ref, gate_ref, w1_ref, w3_ref, w2_ref,
               out_ref, logits_ref, fullw_ref):
    e = pl.program_id(0)
    f = pl.program_id(1)

    @pl.when((e == 0) & (f == 0))
    def _():
        logits = jnp.dot(x_ref[...], gate_ref[...],
                         preferred_element_type=jnp.float32)
        logits_ref[...] = logits
        fullw_ref[...] = _routing(logits)
        out_ref[...] = jnp.zeros_like(out_ref)

    xb = x_ref[...].astype(jnp.bfloat16)
    w1 = w1_ref[0].astype(jnp.bfloat16)
    w3 = w3_ref[0].astype(jnp.bfloat16)
    w2 = w2_ref[0].astype(jnp.bfloat16)
    g = jnp.dot(xb, w1, preferred_element_type=jnp.float32)
    u = jnp.dot(xb, w3, preferred_element_type=jnp.float32)
    h = (g * jax.nn.sigmoid(g)) * u
    y = jnp.dot(h.astype(jnp.bfloat16), w2, preferred_element_type=jnp.float32)
    idx = jax.lax.broadcasted_iota(jnp.int32, (_S, _E), 1)
    w_e = jnp.sum(jnp.where(idx == e, fullw_ref[...], 0.0), axis=-1,
                  keepdims=True)
    out_ref[...] += w_e * y


def kernel(hidden_states, gate_w, w1, w3, w2):
    x = hidden_states.reshape(-1, _D)
    final, logits = pl.pallas_call(
        _moe_dense,
        grid=(_E, _NFT),
        in_specs=[
            pl.BlockSpec((_S, _D), lambda e, f: (0, 0)),
            pl.BlockSpec((_D, _E), lambda e, f: (0, 0)),
            pl.BlockSpec((1, _D, _FT), lambda e, f: (e, 0, f)),
            pl.BlockSpec((1, _D, _FT), lambda e, f: (e, 0, f)),
            pl.BlockSpec((1, _FT, _D), lambda e, f: (e, f, 0)),
        ],
        out_specs=[
            pl.BlockSpec((_S, _D), lambda e, f: (0, 0)),
            pl.BlockSpec((_S, _E), lambda e, f: (0, 0)),
        ],
        out_shape=[
            jax.ShapeDtypeStruct((_S, _D), jnp.float32),
            jax.ShapeDtypeStruct((_S, _E), jnp.float32),
        ],
        scratch_shapes=[pltpu.VMEM((_S, _E), jnp.float32)],
    )(x, gate_w, w1, w3, w2)
    return final.reshape(_B, _S, _D), logits


# speedup vs baseline: 1.3313x; 1.3313x over previous
import jax
import jax.numpy as jnp
from jax.experimental import pallas as pl
from jax.experimental.pallas import tpu as pltpu

_B, _S, _D, _FF, _E = 1, 2048, 1024, 4096, 8
_FT = 1024
_NFT = _FF // _FT
_RB = 512
_NRB = _S // _RB


def _routing(logits):
    probs = jax.nn.softmax(logits, axis=-1)
    idx = jax.lax.broadcasted_iota(jnp.int32, probs.shape, 1)
    m1 = jnp.max(probs, axis=-1, keepdims=True)
    i1 = jnp.min(jnp.where(probs == m1, idx, _E), axis=-1, keepdims=True)
    masked = jnp.where(idx == i1, -1.0, probs)
    m2 = jnp.max(masked, axis=-1, keepdims=True)
    i2 = jnp.min(jnp.where(masked == m2, idx, _E), axis=-1, keepdims=True)
    s = m1 + m2
    return jnp.where(idx == i1, m1 / s, 0.0) + jnp.where(idx == i2, m2 / s, 0.0)


def _moe_dense(x_ref, gate_ref, w1_ref, w3_ref, w2_ref,
               out_ref, logits_ref, fullw_ref):
    e = pl.program_id(0)
    f = pl.program_id(1)
    rb = pl.program_id(2)
    rows = pl.ds(rb * _RB, _RB)

    @pl.when((e == 0) & (f == 0) & (rb == 0))
    def _():
        out_ref[...] = jnp.zeros_like(out_ref)

    @pl.when((e == 0) & (f == 0))
    def _():
        logits = jnp.dot(x_ref[...], gate_ref[...],
                         preferred_element_type=jnp.float32)
        logits_ref[rows, :] = logits
        fullw_ref[rows, :] = _routing(logits)

    xb = x_ref[...].astype(jnp.bfloat16)
    w1 = w1_ref[0].astype(jnp.bfloat16)
    w3 = w3_ref[0].astype(jnp.bfloat16)
    w2 = w2_ref[0].astype(jnp.bfloat16)
    g = jnp.dot(xb, w1, preferred_element_type=jnp.float32)
    u = jnp.dot(xb, w3, preferred_element_type=jnp.float32)
    h = (g * jax.nn.sigmoid(g)) * u
    y = jnp.dot(h.astype(jnp.bfloat16), w2, preferred_element_type=jnp.float32)
    idx = jax.lax.broadcasted_iota(jnp.int32, (_RB, _E), 1)
    w_e = jnp.sum(jnp.where(idx == e, fullw_ref[rows, :], 0.0), axis=-1,
                  keepdims=True)
    out_ref[rows, :] += w_e * y


def kernel(hidden_states, gate_w, w1, w3, w2):
    x = hidden_states.reshape(-1, _D)
    final, logits = pl.pallas_call(
        _moe_dense,
        grid=(_E, _NFT, _NRB),
        in_specs=[
            pl.BlockSpec((_RB, _D), lambda e, f, rb: (rb, 0)),
            pl.BlockSpec((_D, _E), lambda e, f, rb: (0, 0)),
            pl.BlockSpec((1, _D, _FT), lambda e, f, rb: (e, 0, f)),
            pl.BlockSpec((1, _D, _FT), lambda e, f, rb: (e, 0, f)),
            pl.BlockSpec((1, _FT, _D), lambda e, f, rb: (e, f, 0)),
        ],
        out_specs=[
            pl.BlockSpec((_S, _D), lambda e, f, rb: (0, 0)),
            pl.BlockSpec((_S, _E), lambda e, f, rb: (0, 0)),
        ],
        out_shape=[
            jax.ShapeDtypeStruct((_S, _D), jnp.float32),
            jax.ShapeDtypeStruct((_S, _E), jnp.float32),
        ],
        scratch_shapes=[pltpu.VMEM((_S, _E), jnp.float32)],
    )(x, gate_w, w1, w3, w2)
    return final.reshape(_B, _S, _D), logits


# speedup vs baseline: 1.9493x; 1.4643x over previous
import functools

import jax
import jax.numpy as jnp
from jax import lax
from jax.experimental import pallas as pl
from jax.experimental.pallas import tpu as pltpu
from jax.experimental.pallas import tpu_sc as plsc

_B, _S, _D, _FF, _E = 1, 2048, 1024, 4096, 8
_FT = 1024
_NFT = _FF // _FT
_BLK = 512
_SUB = 256
_NBLK = 16
_NROWS = _NBLK * _BLK

_RB = 512
_NRB = _S // _RB

_NC, _NS = 2, 16
_NW = _NC * _NS
_TPW = _S // _NW


def _route_body(x_ref, gate_ref, logits_ref, topw_ref, topi_ref):
    logits = jnp.dot(x_ref[...], gate_ref[...],
                     preferred_element_type=jnp.float32)
    logits_ref[...] = logits
    probs = jax.nn.softmax(logits, axis=-1)
    idx = jax.lax.broadcasted_iota(jnp.int32, probs.shape, 1)
    m1 = jnp.max(probs, axis=-1, keepdims=True)
    i1 = jnp.min(jnp.where(probs == m1, idx, _E), axis=-1, keepdims=True)
    masked = jnp.where(idx == i1, -1.0, probs)
    m2 = jnp.max(masked, axis=-1, keepdims=True)
    i2 = jnp.min(jnp.where(masked == m2, idx, _E), axis=-1, keepdims=True)
    s = m1 + m2
    topw_ref[...] = jnp.concatenate([m1 / s, m2 / s], axis=1)
    topi_ref[...] = jnp.concatenate([i1, i2], axis=1)


def _route(x, gate_w):
    return pl.pallas_call(
        _route_body,
        grid=(_NRB,),
        in_specs=[
            pl.BlockSpec((_RB, _D), lambda rb: (rb, 0)),
            pl.BlockSpec((_D, _E), lambda rb: (0, 0)),
        ],
        out_specs=[
            pl.BlockSpec((_RB, _E), lambda rb: (rb, 0)),
            pl.BlockSpec((_RB, 2), lambda rb: (rb, 0)),
            pl.BlockSpec((_RB, 2), lambda rb: (rb, 0)),
        ],
        out_shape=[
            jax.ShapeDtypeStruct((_S, _E), jnp.float32),
            jax.ShapeDtypeStruct((_S, 2), jnp.float32),
            jax.ShapeDtypeStruct((_S, 2), jnp.int32),
        ],
    )(x, gate_w)


def _dispatch_plan(topi):
    eye = jnp.arange(_E, dtype=jnp.int32)
    sel = ((topi[:, 0:1] == eye) | (topi[:, 1:2] == eye)).astype(jnp.int32)
    selc = jnp.cumsum(sel, axis=0)
    g = selc[-1]
    csum_excl = selc - sel
    nblk_e = (g + _BLK - 1) // _BLK
    blockstart_e = jnp.concatenate(
        [jnp.zeros((1,), jnp.int32), jnp.cumsum(nblk_e)[:-1]])
    rowstart_e = blockstart_e * _BLK
    row_pos = (jnp.take(rowstart_e, topi)
               + jnp.take_along_axis(csum_excl, topi, axis=1))
    used = jnp.sum(nblk_e)
    j = jnp.arange(_NBLK, dtype=jnp.int32)
    jj = jnp.minimum(j, used - 1)
    eb = jnp.sum((jj[:, None] >= blockstart_e[None, :]).astype(jnp.int32),
                 axis=1) - 1
    rib = jnp.where(
        j < used,
        jnp.clip(jnp.take(g, eb) - (jj - jnp.take(blockstart_e, eb)) * _BLK,
                 0, _BLK),
        0)
    idx3 = row_pos.reshape(_NW, _TPW, 2).transpose(0, 2, 1)
    return idx3.astype(jnp.int32), eb, jj, rib.astype(jnp.int32)


def _sc_mesh():
    return plsc.VectorSubcoreMesh(core_axis_name="c", subcore_axis_name="s")


@jax.jit
def _sc_dispatch(x, idx3):
    @functools.partial(
        pl.kernel,
        out_type=jax.ShapeDtypeStruct((_NROWS, _D), jnp.float32),
        mesh=_sc_mesh(),
        scratch_types=[
            pltpu.VMEM((_TPW,), jnp.int32),
            pltpu.VMEM((_TPW,), jnp.int32),
            pltpu.VMEM((_TPW, _D), jnp.float32),
            pltpu.SemaphoreType.DMA,
        ],
    )
    def body(x_hbm, idx_hbm, out_hbm, idxa_v, idxb_v, rows_v, sem):
        wid = lax.axis_index("s") * _NC + lax.axis_index("c")
        base = wid * _TPW
        pltpu.sync_copy(idx_hbm.at[wid, 0], idxa_v)
        pltpu.sync_copy(idx_hbm.at[wid, 1], idxb_v)
        pltpu.async_copy(x_hbm.at[pl.ds(base, _TPW)], rows_v, sem).wait()
        pltpu.async_copy(rows_v, out_hbm.at[idxa_v], sem).wait()
        pltpu.async_copy(rows_v, out_hbm.at[idxb_v], sem).wait()

    return body(x, idx3)


def _mlp_body(eb_ref, sb_ref, rib_ref, xs_ref, w1_ref, w3_ref, w2_ref,
              ys_ref, acc_ref):
    j = pl.program_id(0)
    f = pl.program_id(1)
    nrows = rib_ref[j]

    @pl.when(f == 0)
    def _():
        acc_ref[...] = jnp.zeros_like(acc_ref)

    w1 = w1_ref[0].astype(jnp.bfloat16)
    w3 = w3_ref[0].astype(jnp.bfloat16)
    w2 = w2_ref[0].astype(jnp.bfloat16)
    for s in range(_BLK // _SUB):
        @pl.when(nrows > s * _SUB)
        def _(s=s):
            rows = pl.ds(s * _SUB, _SUB)
            xb = xs_ref[rows, :].astype(jnp.bfloat16)
            g = jnp.dot(xb, w1, preferred_element_type=jnp.float32)
            u = jnp.dot(xb, w3, preferred_element_type=jnp.float32)
            h = (g * jax.nn.sigmoid(g)) * u
            acc_ref[rows, :] += jnp.dot(h.astype(jnp.bfloat16), w2,
                                        preferred_element_type=jnp.float32)

    @pl.when(f == _NFT - 1)
    def _():
        ys_ref[...] = acc_ref[...]


def _mlp(xs, w1, w3, w2, eb, sb, rib):
    grid_spec = pltpu.PrefetchScalarGridSpec(
        num_scalar_prefetch=3,
        grid=(_NBLK, _NFT),
        in_specs=[
            pl.BlockSpec((_BLK, _D), lambda j, f, eb, sb, rib: (sb[j], 0)),
            pl.BlockSpec((1, _D, _FT), lambda j, f, eb, sb, rib: (eb[j], 0, f)),
            pl.BlockSpec((1, _D, _FT), lambda j, f, eb, sb, rib: (eb[j], 0, f)),
            pl.BlockSpec((1, _FT, _D), lambda j, f, eb, sb, rib: (eb[j], f, 0)),
        ],
        out_specs=pl.BlockSpec((_BLK, _D), lambda j, f, eb, sb, rib: (j, 0)),
        scratch_shapes=[pltpu.VMEM((_BLK, _D), jnp.float32)],
    )
    return pl.pallas_call(
        _mlp_body,
        grid_spec=grid_spec,
        out_shape=jax.ShapeDtypeStruct((_NROWS, _D), jnp.float32),
    )(eb, sb, rib, xs, w1, w3, w2)


@jax.jit
def _sc_combine_gather(ys, idx3):
    @functools.partial(
        pl.kernel,
        out_type=jax.ShapeDtypeStruct((2, _S, _D), jnp.float32),
        mesh=_sc_mesh(),
        scratch_types=[
            pltpu.VMEM((_TPW,), jnp.int32),
            pltpu.VMEM((_TPW,), jnp.int32),
            pltpu.VMEM((_TPW, _D), jnp.float32),
            pltpu.SemaphoreType.DMA,
        ],
    )
    def body(ys_hbm, idx_hbm, out_hbm, idxa_v, idxb_v, rows_v, sem):
        wid = lax.axis_index("s") * _NC + lax.axis_index("c")
        base = wid * _TPW
        pltpu.sync_copy(idx_hbm.at[wid, 0], idxa_v)
        pltpu.sync_copy(idx_hbm.at[wid, 1], idxb_v)
        pltpu.async_copy(ys_hbm.at[idxa_v], rows_v, sem).wait()
        pltpu.async_copy(rows_v, out_hbm.at[0, pl.ds(base, _TPW)], sem).wait()
        pltpu.async_copy(ys_hbm.at[idxb_v], rows_v, sem).wait()
        pltpu.async_copy(rows_v, out_hbm.at[1, pl.ds(base, _TPW)], sem).wait()

    return body(ys, idx3)


def _combine_body(ya_ref, yb_ref, topw_ref, out_ref):
    w = topw_ref[...]
    out_ref[...] = w[:, 0:1] * ya_ref[0] + w[:, 1:2] * yb_ref[0]


def _combine(yab, topw):
    return pl.pallas_call(
        _combine_body,
        grid=(_NRB,),
        in_specs=[
            pl.BlockSpec((1, _RB, _D), lambda rb: (0, rb, 0)),
            pl.BlockSpec((1, _RB, _D), lambda rb: (1, rb, 0)),
            pl.BlockSpec((_RB, 2), lambda rb: (rb, 0)),
        ],
        out_specs=pl.BlockSpec((_RB, _D), lambda rb: (rb, 0)),
        out_shape=jax.ShapeDtypeStruct((_S, _D), jnp.float32),
    )(yab, yab, topw)


def kernel(hidden_states, gate_w, w1, w3, w2):
    x = hidden_states.reshape(-1, _D)
    logits, topw, topi = _route(x, gate_w)
    idx3, eb, sb, rib = _dispatch_plan(topi)
    xs = _sc_dispatch(x, idx3)
    ys = _mlp(xs, w1, w3, w2, eb, sb, rib)
    yab = _sc_combine_gather(ys, idx3)
    final = _combine(yab, topw)
    return final.reshape(_B, _S, _D), logits


# speedup vs baseline: 2.1506x; 1.1033x over previous
import functools

import jax
import jax.numpy as jnp
from jax import lax
from jax.experimental import pallas as pl
from jax.experimental.pallas import tpu as pltpu
from jax.experimental.pallas import tpu_sc as plsc

_B, _S, _D, _FF, _E = 1, 2048, 1024, 4096, 8
_FT = 1024
_NFT = _FF // _FT
_BLK = 1024
_SUB = 256
_NBLK = 12
_NROWS = _NBLK * _BLK

_RB = 512
_NRB = _S // _RB

_NC, _NS = 2, 16
_NW = _NC * _NS
_TPW = _S // _NW


def _route_body(x_ref, gate_ref, logits_ref, topw_ref, topi_ref):
    logits = jnp.dot(x_ref[...], gate_ref[...],
                     preferred_element_type=jnp.float32)
    logits_ref[...] = logits
    probs = jax.nn.softmax(logits, axis=-1)
    idx = jax.lax.broadcasted_iota(jnp.int32, probs.shape, 1)
    m1 = jnp.max(probs, axis=-1, keepdims=True)
    i1 = jnp.min(jnp.where(probs == m1, idx, _E), axis=-1, keepdims=True)
    masked = jnp.where(idx == i1, -1.0, probs)
    m2 = jnp.max(masked, axis=-1, keepdims=True)
    i2 = jnp.min(jnp.where(masked == m2, idx, _E), axis=-1, keepdims=True)
    s = m1 + m2
    topw_ref[...] = jnp.concatenate([m1 / s, m2 / s], axis=1)
    topi_ref[...] = jnp.concatenate([i1, i2], axis=1)


def _route(x, gate_w):
    return pl.pallas_call(
        _route_body,
        grid=(_NRB,),
        in_specs=[
            pl.BlockSpec((_RB, _D), lambda rb: (rb, 0)),
            pl.BlockSpec((_D, _E), lambda rb: (0, 0)),
        ],
        out_specs=[
            pl.BlockSpec((_RB, _E), lambda rb: (rb, 0)),
            pl.BlockSpec((_RB, 2), lambda rb: (rb, 0)),
            pl.BlockSpec((_RB, 2), lambda rb: (rb, 0)),
        ],
        out_shape=[
            jax.ShapeDtypeStruct((_S, _E), jnp.float32),
            jax.ShapeDtypeStruct((_S, 2), jnp.float32),
            jax.ShapeDtypeStruct((_S, 2), jnp.int32),
        ],
    )(x, gate_w)


def _dispatch_plan(topi):
    eye = jnp.arange(_E, dtype=jnp.int32)
    sel = ((topi[:, 0:1] == eye) | (topi[:, 1:2] == eye)).astype(jnp.int32)
    selc = jnp.cumsum(sel, axis=0)
    g = selc[-1]
    csum_excl = selc - sel
    nblk_e = (g + _BLK - 1) // _BLK
    blockstart_e = jnp.concatenate(
        [jnp.zeros((1,), jnp.int32), jnp.cumsum(nblk_e)[:-1]])
    rowstart_e = blockstart_e * _BLK
    row_pos = (jnp.take(rowstart_e, topi)
               + jnp.take_along_axis(csum_excl, topi, axis=1))
    used = jnp.sum(nblk_e)
    j = jnp.arange(_NBLK, dtype=jnp.int32)
    jj = jnp.minimum(j, used - 1)
    eb = jnp.sum((jj[:, None] >= blockstart_e[None, :]).astype(jnp.int32),
                 axis=1) - 1
    rib = jnp.where(
        j < used,
        jnp.clip(jnp.take(g, eb) - (jj - jnp.take(blockstart_e, eb)) * _BLK,
                 0, _BLK),
        0)
    idx3 = row_pos.reshape(_NW, _TPW, 2).transpose(0, 2, 1)
    return idx3.astype(jnp.int32), eb, jj, rib.astype(jnp.int32)


def _sc_mesh():
    return plsc.VectorSubcoreMesh(core_axis_name="c", subcore_axis_name="s")


@jax.jit
def _sc_dispatch(x, idx3):
    @functools.partial(
        pl.kernel,
        out_type=jax.ShapeDtypeStruct((_NROWS, _D), jnp.float32),
        mesh=_sc_mesh(),
        scratch_types=[
            pltpu.VMEM((_TPW,), jnp.int32),
            pltpu.VMEM((_TPW,), jnp.int32),
            pltpu.VMEM((_TPW, _D), jnp.float32),
            pltpu.SemaphoreType.DMA,
        ],
    )
    def body(x_hbm, idx_hbm, out_hbm, idxa_v, idxb_v, rows_v, sem):
        wid = lax.axis_index("s") * _NC + lax.axis_index("c")
        base = wid * _TPW
        pltpu.sync_copy(idx_hbm.at[wid, 0], idxa_v)
        pltpu.sync_copy(idx_hbm.at[wid, 1], idxb_v)
        pltpu.async_copy(x_hbm.at[pl.ds(base, _TPW)], rows_v, sem).wait()
        pltpu.async_copy(rows_v, out_hbm.at[idxa_v], sem).wait()
        pltpu.async_copy(rows_v, out_hbm.at[idxb_v], sem).wait()

    return body(x, idx3)


def _mlp_body(eb_ref, sb_ref, rib_ref, xs_ref, w1_ref, w3_ref, w2_ref,
              ys_ref, acc_ref):
    j = pl.program_id(0)
    f = pl.program_id(1)
    nrows = rib_ref[j]

    @pl.when(f == 0)
    def _():
        acc_ref[...] = jnp.zeros_like(acc_ref)

    w1 = w1_ref[0].astype(jnp.bfloat16)
    w3 = w3_ref[0].astype(jnp.bfloat16)
    w2 = w2_ref[0].astype(jnp.bfloat16)
    for s in range(_BLK // _SUB):
        @pl.when(nrows > s * _SUB)
        def _(s=s):
            rows = pl.ds(s * _SUB, _SUB)
            xb = xs_ref[rows, :].astype(jnp.bfloat16)
            g = jnp.dot(xb, w1, preferred_element_type=jnp.float32)
            u = jnp.dot(xb, w3, preferred_element_type=jnp.float32)
            h = (g * jax.nn.sigmoid(g)) * u
            acc_ref[rows, :] += jnp.dot(h.astype(jnp.bfloat16), w2,
                                        preferred_element_type=jnp.float32)

    @pl.when((f == _NFT - 1) & (nrows > 0))
    def _():
        ys_ref[...] = acc_ref[...]


def _mlp(xs, w1, w3, w2, eb, sb, rib):
    grid_spec = pltpu.PrefetchScalarGridSpec(
        num_scalar_prefetch=3,
        grid=(_NBLK, _NFT),
        in_specs=[
            pl.BlockSpec((_BLK, _D), lambda j, f, eb, sb, rib: (sb[j], 0)),
            pl.BlockSpec((1, _D, _FT), lambda j, f, eb, sb, rib: (eb[j], 0, f)),
            pl.BlockSpec((1, _D, _FT), lambda j, f, eb, sb, rib: (eb[j], 0, f)),
            pl.BlockSpec((1, _FT, _D), lambda j, f, eb, sb, rib: (eb[j], f, 0)),
        ],
        out_specs=pl.BlockSpec((_BLK, _D), lambda j, f, eb, sb, rib: (sb[j], 0)),
        scratch_shapes=[pltpu.VMEM((_BLK, _D), jnp.float32)],
    )
    return pl.pallas_call(
        _mlp_body,
        grid_spec=grid_spec,
        out_shape=jax.ShapeDtypeStruct((_NROWS, _D), jnp.float32),
    )(eb, sb, rib, xs, w1, w3, w2)


@jax.jit
def _sc_combine_gather(ys, idx3):
    @functools.partial(
        pl.kernel,
        out_type=jax.ShapeDtypeStruct((2, _S, _D), jnp.float32),
        mesh=_sc_mesh(),
        scratch_types=[
            pltpu.VMEM((_TPW,), jnp.int32),
            pltpu.VMEM((_TPW,), jnp.int32),
            pltpu.VMEM((_TPW, _D), jnp.float32),
            pltpu.SemaphoreType.DMA,
        ],
    )
    def body(ys_hbm, idx_hbm, out_hbm, idxa_v, idxb_v, rows_v, sem):
        wid = lax.axis_index("s") * _NC + lax.axis_index("c")
        base = wid * _TPW
        pltpu.sync_copy(idx_hbm.at[wid, 0], idxa_v)
        pltpu.sync_copy(idx_hbm.at[wid, 1], idxb_v)
        pltpu.async_copy(ys_hbm.at[idxa_v], rows_v, sem).wait()
        pltpu.async_copy(rows_v, out_hbm.at[0, pl.ds(base, _TPW)], sem).wait()
        pltpu.async_copy(ys_hbm.at[idxb_v], rows_v, sem).wait()
        pltpu.async_copy(rows_v, out_hbm.at[1, pl.ds(base, _TPW)], sem).wait()

    return body(ys, idx3)


def _combine_body(ya_ref, yb_ref, topw_ref, out_ref):
    w = topw_ref[...]
    out_ref[...] = w[:, 0:1] * ya_ref[0] + w[:, 1:2] * yb_ref[0]


def _combine(yab, topw):
    return pl.pallas_call(
        _combine_body,
        grid=(_NRB,),
        in_specs=[
            pl.BlockSpec((1, _RB, _D), lambda rb: (0, rb, 0)),
            pl.BlockSpec((1, _RB, _D), lambda rb: (1, rb, 0)),
            pl.BlockSpec((_RB, 2), lambda rb: (rb, 0)),
        ],
        out_specs=pl.BlockSpec((_RB, _D), lambda rb: (rb, 0)),
        out_shape=jax.ShapeDtypeStruct((_S, _D), jnp.float32),
    )(yab, yab, topw)


def kernel(hidden_states, gate_w, w1, w3, w2):
    x = hidden_states.reshape(-1, _D)
    logits, topw, topi = _route(x, gate_w)
    idx3, eb, sb, rib = _dispatch_plan(topi)
    xs = _sc_dispatch(x, idx3)
    ys = _mlp(xs, w1, w3, w2, eb, sb, rib)
    yab = _sc_combine_gather(ys, idx3)
    final = _combine(yab, topw)
    return final.reshape(_B, _S, _D), logits


# speedup vs baseline: 2.4420x; 1.1355x over previous
import functools

import jax
import jax.numpy as jnp
from jax import lax
from jax.experimental import pallas as pl
from jax.experimental.pallas import tpu as pltpu
from jax.experimental.pallas import tpu_sc as plsc

_B, _S, _D, _FF, _E = 1, 2048, 1024, 4096, 8
_FT = 1024
_NFT = _FF // _FT
_BLK = 1024
_SUB = 256
_NBLK = 12
_NROWS = _NBLK * _BLK

_RB = 512
_NRB = _S // _RB

_NC, _NS = 2, 16
_NW = _NC * _NS
_TPW = _S // _NW


def _route_body(x_ref, gate_ref, logits_ref, topw_ref, topi_ref):
    logits = jnp.dot(x_ref[...], gate_ref[...],
                     preferred_element_type=jnp.float32)
    logits_ref[...] = logits
    probs = jax.nn.softmax(logits, axis=-1)
    idx = jax.lax.broadcasted_iota(jnp.int32, probs.shape, 1)
    m1 = jnp.max(probs, axis=-1, keepdims=True)
    i1 = jnp.min(jnp.where(probs == m1, idx, _E), axis=-1, keepdims=True)
    masked = jnp.where(idx == i1, -1.0, probs)
    m2 = jnp.max(masked, axis=-1, keepdims=True)
    i2 = jnp.min(jnp.where(masked == m2, idx, _E), axis=-1, keepdims=True)
    s = m1 + m2
    topw_ref[...] = jnp.concatenate([m1 / s, m2 / s], axis=1)
    topi_ref[...] = jnp.concatenate([i1, i2], axis=1)


def _route(x, gate_w):
    return pl.pallas_call(
        _route_body,
        grid=(_NRB,),
        in_specs=[
            pl.BlockSpec((_RB, _D), lambda rb: (rb, 0)),
            pl.BlockSpec((_D, _E), lambda rb: (0, 0)),
        ],
        out_specs=[
            pl.BlockSpec((_RB, _E), lambda rb: (rb, 0)),
            pl.BlockSpec((_RB, 2), lambda rb: (rb, 0)),
            pl.BlockSpec((_RB, 2), lambda rb: (rb, 0)),
        ],
        out_shape=[
            jax.ShapeDtypeStruct((_S, _E), jnp.float32),
            jax.ShapeDtypeStruct((_S, 2), jnp.float32),
            jax.ShapeDtypeStruct((_S, 2), jnp.int32),
        ],
    )(x, gate_w)


def _dispatch_plan(topi):
    eye = jnp.arange(_E, dtype=jnp.int32)
    sel = ((topi[:, 0:1] == eye) | (topi[:, 1:2] == eye)).astype(jnp.int32)
    selc = jnp.cumsum(sel, axis=0)
    g = selc[-1]
    csum_excl = selc - sel
    nblk_e = (g + _BLK - 1) // _BLK
    blockstart_e = jnp.concatenate(
        [jnp.zeros((1,), jnp.int32), jnp.cumsum(nblk_e)[:-1]])
    rowstart_e = blockstart_e * _BLK
    row_pos = (jnp.take(rowstart_e, topi)
               + jnp.take_along_axis(csum_excl, topi, axis=1))
    used = jnp.sum(nblk_e)
    j = jnp.arange(_NBLK, dtype=jnp.int32)
    jj = jnp.minimum(j, used - 1)
    eb = jnp.sum((jj[:, None] >= blockstart_e[None, :]).astype(jnp.int32),
                 axis=1) - 1
    rib = jnp.where(
        j < used,
        jnp.clip(jnp.take(g, eb) - (jj - jnp.take(blockstart_e, eb)) * _BLK,
                 0, _BLK),
        0)
    idx3 = row_pos.reshape(_NW, _TPW, 2).transpose(0, 2, 1)
    return idx3.astype(jnp.int32), eb, jj, rib.astype(jnp.int32)


def _sc_mesh():
    return plsc.VectorSubcoreMesh(core_axis_name="c", subcore_axis_name="s")


@jax.jit
def _sc_dispatch(x, idx3):
    @functools.partial(
        pl.kernel,
        out_type=jax.ShapeDtypeStruct((_NROWS, _D), jnp.float32),
        mesh=_sc_mesh(),
        scratch_types=[
            pltpu.VMEM((_TPW,), jnp.int32),
            pltpu.VMEM((_TPW,), jnp.int32),
            pltpu.VMEM((_TPW, _D), jnp.float32),
            pltpu.SemaphoreType.DMA,
        ],
    )
    def body(x_hbm, idx_hbm, out_hbm, idxa_v, idxb_v, rows_v, sem):
        wid = lax.axis_index("s") * _NC + lax.axis_index("c")
        base = wid * _TPW
        pltpu.sync_copy(idx_hbm.at[wid, 0], idxa_v)
        pltpu.sync_copy(idx_hbm.at[wid, 1], idxb_v)
        pltpu.async_copy(x_hbm.at[pl.ds(base, _TPW)], rows_v, sem).wait()
        pltpu.async_copy(rows_v, out_hbm.at[idxa_v], sem).wait()
        pltpu.async_copy(rows_v, out_hbm.at[idxb_v], sem).wait()

    return body(x, idx3)


def _mlp_body(eb_ref, sb_ref, rib_ref, xs_ref, w1_ref, w3_ref, w2_ref,
              ys_ref, acc_ref):
    j = pl.program_id(0)
    f = pl.program_id(1)
    nrows = rib_ref[j]

    w1 = w1_ref[0].astype(jnp.bfloat16)
    w3 = w3_ref[0].astype(jnp.bfloat16)
    w2 = w2_ref[0].astype(jnp.bfloat16)
    for s in range(_BLK // _SUB):
        @pl.when(nrows > s * _SUB)
        def _(s=s):
            rows = pl.ds(s * _SUB, _SUB)
            xb = xs_ref[rows, :].astype(jnp.bfloat16)
            g = jnp.dot(xb, w1, preferred_element_type=jnp.float32)
            u = jnp.dot(xb, w3, preferred_element_type=jnp.float32)
            h = (g * jax.nn.sigmoid(g)) * u
            y = jnp.dot(h.astype(jnp.bfloat16), w2,
                        preferred_element_type=jnp.float32)

            @pl.when(f == 0)
            def _():
                acc_ref[rows, :] = y

            @pl.when(f != 0)
            def _():
                acc_ref[rows, :] += y

    @pl.when((f == _NFT - 1) & (nrows > 0))
    def _():
        ys_ref[...] = acc_ref[...]


def _mlp(xs, w1, w3, w2, eb, sb, rib):
    grid_spec = pltpu.PrefetchScalarGridSpec(
        num_scalar_prefetch=3,
        grid=(_NBLK, _NFT),
        in_specs=[
            pl.BlockSpec((_BLK, _D), lambda j, f, eb, sb, rib: (sb[j], 0)),
            pl.BlockSpec(
                (1, _D, _FT),
                lambda j, f, eb, sb, rib:
                    (eb[j], 0, jnp.where(rib[j] > 0, f, _NFT - 1))),
            pl.BlockSpec(
                (1, _D, _FT),
                lambda j, f, eb, sb, rib:
                    (eb[j], 0, jnp.where(rib[j] > 0, f, _NFT - 1))),
            pl.BlockSpec(
                (1, _FT, _D),
                lambda j, f, eb, sb, rib:
                    (eb[j], jnp.where(rib[j] > 0, f, _NFT - 1), 0)),
        ],
        out_specs=pl.BlockSpec((_BLK, _D), lambda j, f, eb, sb, rib: (sb[j], 0)),
        scratch_shapes=[pltpu.VMEM((_BLK, _D), jnp.float32)],
    )
    return pl.pallas_call(
        _mlp_body,
        grid_spec=grid_spec,
        out_shape=jax.ShapeDtypeStruct((_NROWS, _D), jnp.float32),
    )(eb, sb, rib, xs, w1, w3, w2)


@jax.jit
def _sc_combine_gather(ys, idx3):
    @functools.partial(
        pl.kernel,
        out_type=jax.ShapeDtypeStruct((2, _S, _D), jnp.float32),
        mesh=_sc_mesh(),
        scratch_types=[
            pltpu.VMEM((_TPW,), jnp.int32),
            pltpu.VMEM((_TPW,), jnp.int32),
            pltpu.VMEM((_TPW, _D), jnp.float32),
            pltpu.SemaphoreType.DMA,
        ],
    )
    def body(ys_hbm, idx_hbm, out_hbm, idxa_v, idxb_v, rows_v, sem):
        wid = lax.axis_index("s") * _NC + lax.axis_index("c")
        base = wid * _TPW
        pltpu.sync_copy(idx_hbm.at[wid, 0], idxa_v)
        pltpu.sync_copy(idx_hbm.at[wid, 1], idxb_v)
        pltpu.async_copy(ys_hbm.at[idxa_v], rows_v, sem).wait()
        pltpu.async_copy(rows_v, out_hbm.at[0, pl.ds(base, _TPW)], sem).wait()
        pltpu.async_copy(ys_hbm.at[idxb_v], rows_v, sem).wait()
        pltpu.async_copy(rows_v, out_hbm.at[1, pl.ds(base, _TPW)], sem).wait()

    return body(ys, idx3)


def _combine_body(ya_ref, yb_ref, topw_ref, out_ref):
    w = topw_ref[...]
    out_ref[...] = w[:, 0:1] * ya_ref[0] + w[:, 1:2] * yb_ref[0]


def _combine(yab, topw):
    return pl.pallas_call(
        _combine_body,
        grid=(_NRB,),
        in_specs=[
            pl.BlockSpec((1, _RB, _D), lambda rb: (0, rb, 0)),
            pl.BlockSpec((1, _RB, _D), lambda rb: (1, rb, 0)),
            pl.BlockSpec((_RB, 2), lambda rb: (rb, 0)),
        ],
        out_specs=pl.BlockSpec((_RB, _D), lambda rb: (rb, 0)),
        out_shape=jax.ShapeDtypeStruct((_S, _D), jnp.float32),
    )(yab, yab, topw)


def kernel(hidden_states, gate_w, w1, w3, w2):
    x = hidden_states.reshape(-1, _D)
    logits, topw, topi = _route(x, gate_w)
    idx3, eb, sb, rib = _dispatch_plan(topi)
    xs = _sc_dispatch(x, idx3)
    ys = _mlp(xs, w1, w3, w2, eb, sb, rib)
    yab = _sc_combine_gather(ys, idx3)
    final = _combine(yab, topw)
    return final.reshape(_B, _S, _D), logits
